# baseline (speedup 1.0000x reference)
import jax
import jax.numpy as jnp
from jax.experimental import pallas as pl

_N, _T, _P, _D = 8, 48, 325, 512
_M = _N * _T * _P

_BM = 1920
_GRID = _M // _BM


def _body(x_ref, mon_ref, b_ref, tbvb_ref, tbet_ref, tbut_ref, out_ref):
    xb = x_ref[...]
    bad = jnp.isnan(xb)
    xc = jnp.where(bad, 0.0, xb).astype(jnp.bfloat16)
    y = jax.lax.dot_general(xc, b_ref[...], (((1,), (0,)), ((), ())),
                            preferred_element_type=jnp.float32)

    monb = mon_ref[...] > 0.5
    m_emp = jnp.broadcast_to(bad[:, 0:1] & monb, (_BM, _D))
    m_un = jnp.broadcast_to(~monb, (_BM, _D))
    tbvb = jnp.broadcast_to(tbvb_ref[...], (_BM, _D))
    tbet = jnp.broadcast_to(tbet_ref[...], (_BM, _D))
    tbut = jnp.broadcast_to(tbut_ref[...], (_BM, _D))
    bias = jnp.where(m_un, tbut, jnp.where(m_emp, tbet, tbvb))
    out_ref[...] = y + bias


def kernel(x, monitor_mask, time_emb_w, time_emb_b, value_emb_w, value_emb_b,
           empty_token, unmonitored_token):
    value = jnp.where(monitor_mask, x[..., 0], jnp.nan)
    xm = jnp.stack([value, x[..., 1]], axis=-1).reshape(_M, 2)
    mon = monitor_mask.reshape(_M, 1).astype(jnp.float32)

    bmat = jnp.concatenate([value_emb_w.reshape(1, _D),
                            time_emb_w.reshape(1, _D)], axis=0
                           ).astype(jnp.bfloat16)
    tb = time_emb_b.reshape(1, _D)
    tbvb = tb + value_emb_b.reshape(1, _D)
    tbet = tb + empty_token.reshape(1, _D)
    tbut = tb + unmonitored_token.reshape(1, _D)

    full = pl.BlockSpec((1, _D), lambda i: (0, 0))
    out = pl.pallas_call(
        _body,
        grid=(_GRID,),
        in_specs=[pl.BlockSpec((_BM, 2), lambda i: (i, 0)),
                  pl.BlockSpec((_BM, 1), lambda i: (i, 0)),
                  pl.BlockSpec((2, _D), lambda i: (0, 0)),
                  full, full, full],
        out_specs=pl.BlockSpec((_BM, _D), lambda i: (i, 0)),
        out_shape=jax.ShapeDtypeStruct((_M, _D), jnp.float32),
    )(xm, mon, bmat, tbvb, tbet, tbut)
    return out.reshape(_N, _T, _P, _D)


# speedup vs baseline: 1.1607x; 1.1607x over previous
import jax
import jax.numpy as jnp
from jax.experimental import pallas as pl

_N, _T, _P, _D = 8, 48, 325, 512
_M = _N * _T * _P

_BM = 1920
_GRID = _M // _BM


def _body(a_ref, b_ref, out_ref):
    a = a_ref[...].astype(jnp.bfloat16)
    out_ref[...] = jax.lax.dot_general(
        a, b_ref[...], (((0,), (0,)), ((), ())),
        preferred_element_type=jnp.float32)


def kernel(x, monitor_mask, time_emb_w, time_emb_b, value_emb_w, value_emb_b,
           empty_token, unmonitored_token):
    f32 = jnp.float32
    v = x[..., 0].reshape(_M)
    t = x[..., 1].reshape(_M)
    mon = monitor_mask.reshape(_M)
    bad = jnp.isnan(v)
    p_valid = mon & ~bad
    coef = jnp.where(p_valid, v, 0.0)
    zeros = jnp.zeros((_M,), f32)
    a = jnp.stack([t, coef, p_valid.astype(f32), (mon & bad).astype(f32),
                   (~mon).astype(f32), jnp.ones((_M,), f32), zeros, zeros],
                  axis=0)

    b = jnp.concatenate([time_emb_w.reshape(1, _D),
                         value_emb_w.reshape(1, _D),
                         value_emb_b.reshape(1, _D),
                         empty_token.reshape(1, _D),
                         unmonitored_token.reshape(1, _D),
                         time_emb_b.reshape(1, _D),
                         jnp.zeros((2, _D), f32)], axis=0
                        ).astype(jnp.bfloat16)

    out = pl.pallas_call(
        _body,
        grid=(_GRID,),
        in_specs=[pl.BlockSpec((8, _BM), lambda i: (0, i)),
                  pl.BlockSpec((8, _D), lambda i: (0, 0))],
        out_specs=pl.BlockSpec((_BM, _D), lambda i: (i, 0)),
        out_shape=jax.ShapeDtypeStruct((_M, _D), jnp.float32),
    )(a, b)
    return out.reshape(_N, _T, _P, _D)
